# baseline (speedup 1.0000x reference)
import functools
import math

import jax
import jax.numpy as jnp
from jax import lax
from jax.experimental import pallas as pl
from jax.experimental.pallas import tpu as pltpu
from jax.experimental.pallas import tpu_sc as plsc

_B = 2
_L = 512
_K = 32
_NATOMS = 5
_NPAIR = _NATOMS * _NATOMS
_NBINS = 37
_DWE = _NPAIR * _NBINS
_DOUT = 64
_NEDGE = _B * _L * _K
_NROWS = _B * _L

_NW = 32
_EPW = _NEDGE // _NW
_CHUNK = 256
_GPC = _CHUNK // 16
_NCHUNK = _EPW // _CHUNK

_LN10K = math.log(10000.0) / 64.0


def _prep_body(x_ref, w_ref, b_ref, ac_ref, we_ref, pew_ref):
    x = x_ref[...]

    def norm3(v):
        n = jnp.sqrt(jnp.sum(v * v, axis=0, keepdims=True))
        return v / jnp.maximum(n, 1e-12)

    n_at = x[0:3]
    ca = x[3:6]
    c_at = x[6:9]
    ca_n = norm3(n_at - ca)
    ca_c = norm3(c_at - ca)
    bis = norm3(ca_n + ca_c)
    perp = norm3(
        jnp.stack(
            [
                ca_n[1] * ca_c[2] - ca_n[2] * ca_c[1],
                ca_n[2] * ca_c[0] - ca_n[0] * ca_c[2],
                ca_n[0] * ca_c[1] - ca_n[1] * ca_c[0],
            ],
            axis=0,
        )
    )
    cb_dir = norm3(-bis + 0.5 * perp)
    vcb = ca + 1.54 * cb_dir
    ac_ref[...] = jnp.concatenate([x, vcb], axis=0)

    we_ref[...] = w_ref[0:_DWE, :]

    pos = lax.broadcasted_iota(jnp.int32, (65, 64), 0).astype(jnp.float32) - 32.0
    col = lax.broadcasted_iota(jnp.int32, (65, 64), 1)
    freq = ((col >> 1) * 2).astype(jnp.float32)
    ang = pos * jnp.exp(freq * (-_LN10K))
    pe = jnp.where((col % 2) == 0, jnp.sin(ang), jnp.cos(ang))
    pew_ref[...] = (
        jnp.dot(pe, w_ref[_DWE:, :], preferred_element_type=jnp.float32)
        + b_ref[...]
    )


def _sc_edges(coords_hbm, we_hbm, pew_hbm, nbr_hbm, out_hbm,
              coords_v, t_v, nbr_v, out_v, rb_v):
    wid = lax.axis_index("s") * 2 + lax.axis_index("c")
    ebase = wid * _EPW

    pltpu.sync_copy(coords_hbm, coords_v.at[pl.ds(0, 15 * _NROWS)])
    pltpu.sync_copy(we_hbm, t_v.at[pl.ds(0, _DWE * _DOUT)])
    pltpu.sync_copy(pew_hbm, t_v.at[pl.ds(_DWE * _DOUT, 65 * _DOUT)])
    pltpu.sync_copy(nbr_hbm.at[pl.ds(ebase, _EPW)], nbr_v)

    iota16 = lax.iota(jnp.int32, 16)
    b_id = ebase >> 14
    magic = jnp.int32(0x5F3759DF)

    def group_body(g, carry_g):
        e0 = ebase + g * 16
        l_id = (e0 >> 5) & (_L - 1)
        cb_row = (b_id << 9) + l_id
        nv = nbr_v[pl.ds(g * 16, 16)]
        n_row = nv + (b_id << 9)

        cc = [
            jnp.full((16,), coords_v[pl.ds(a * _NROWS + cb_row, 16)][0],
                     jnp.float32)
            for a in range(15)
        ]
        cn = [
            plsc.load_gather(coords_v, [n_row + a * _NROWS])
            for a in range(15)
        ]

        rbs = []
        for ai in range(_NATOMS):
            for aj in range(_NATOMS):
                p = ai * _NATOMS + aj
                dx = cc[3 * ai + 0] - cn[3 * aj + 0]
                dy = cc[3 * ai + 1] - cn[3 * aj + 1]
                dz = cc[3 * ai + 2] - cn[3 * aj + 2]
                s = dx * dx + dy * dy + dz * dz
                s = jnp.maximum(s, 1.0)
                yi = magic - (lax.bitcast_convert_type(s, jnp.int32) >> 1)
                y = lax.bitcast_convert_type(yi, jnp.float32)
                h = s * 0.5
                y = y * (1.5 - h * y * y)
                y = y * (1.5 - h * y * y)
                y = y * (1.5 - h * y * y)
                d = s * y
                xf = 2.0 * d - 4.0
                t = xf.astype(jnp.int32)
                ceil = jnp.where(xf > t.astype(jnp.float32), t + 1, t)
                binp = jnp.clip(ceil, 0, _NBINS - 1)
                rbs.append((binp + p * _NBINS) << 6)
        rel = jnp.clip(nv - l_id, -32, 32)
        rbs.append((rel + (_DWE + 32)) << 6)

        tr_idx = iota16 * 33
        for r in range(26):
            plsc.store_scatter(rb_v, [tr_idx + r], rbs[r])

        obase = (g % _GPC) * 16 * _DOUT

        def edge_body(i, carry):
            for e_sub in range(2):
                le = i * 2 + e_sub
                ob = obase + le * _DOUT
                v0 = rb_v[pl.ds(le * 33, 16)]
                v1 = rb_v[pl.ds(le * 33 + 16, 16)]
                rows = [v0[r] for r in range(16)] + [v1[r] for r in range(10)]
                for u in range(4):
                    vals = [t_v[pl.ds(rw + 16 * u, 16)] for rw in rows]
                    while len(vals) > 1:
                        vals = [
                            vals[2 * t] + vals[2 * t + 1]
                            for t in range(len(vals) // 2)
                        ] + ([vals[-1]] if len(vals) % 2 else [])
                    out_v[pl.ds(ob + 16 * u, 16)] = vals[0]
            return carry

        lax.fori_loop(0, 8, edge_body, 0)
        return carry_g

    def chunk_body(ci, carry_c):
        lax.fori_loop(ci * _GPC, (ci + 1) * _GPC, group_body, 0)
        off = (ebase + ci * _CHUNK) * _DOUT
        pltpu.sync_copy(out_v, out_hbm.at[pl.ds(off, _CHUNK * _DOUT)])
        return carry_c

    lax.fori_loop(0, _NCHUNK, chunk_body, 0)


_sc_call = functools.partial(
    pl.kernel,
    mesh=plsc.VectorSubcoreMesh(core_axis_name="c", subcore_axis_name="s"),
    out_type=jax.ShapeDtypeStruct((_NEDGE * _DOUT,), jnp.float32),
    compiler_params=pltpu.CompilerParams(needs_layout_passes=False),
    scratch_types=[
        pltpu.VMEM((15 * _NROWS + 16,), jnp.float32),
        pltpu.VMEM(((_DWE + 65) * _DOUT,), jnp.float32),
        pltpu.VMEM((_EPW,), jnp.int32),
        pltpu.VMEM((_CHUNK * _DOUT,), jnp.float32),
        pltpu.VMEM((16 * 33,), jnp.int32),
    ],
)(_sc_edges)


def kernel(coordinates, neighbor_indices, W_edge, b_edge):
    B, L, _, _ = coordinates.shape
    k = neighbor_indices.shape[-1]
    x_t = coordinates.transpose(2, 3, 0, 1).reshape(12, B * L)
    ac, we_t, pew_t = pl.pallas_call(
        _prep_body,
        out_shape=[
            jax.ShapeDtypeStruct((15, B * L), jnp.float32),
            jax.ShapeDtypeStruct((_DWE, _DOUT), jnp.float32),
            jax.ShapeDtypeStruct((65, _DOUT), jnp.float32),
        ],
    )(x_t, W_edge, b_edge.reshape(1, _DOUT))
    out_flat = _sc_call(
        ac.reshape(-1),
        we_t.reshape(-1),
        pew_t.reshape(-1),
        neighbor_indices.reshape(-1).astype(jnp.int32),
    )
    return out_flat.reshape(B, L, k, _DOUT)


# speedup vs baseline: 1.1782x; 1.1782x over previous
import functools
import math

import jax
import jax.numpy as jnp
from jax import lax
from jax.experimental import pallas as pl
from jax.experimental.pallas import tpu as pltpu
from jax.experimental.pallas import tpu_sc as plsc

_B = 2
_L = 512
_K = 32
_NATOMS = 5
_NPAIR = _NATOMS * _NATOMS
_NBINS = 37
_DWE = _NPAIR * _NBINS
_DOUT = 64
_NEDGE = _B * _L * _K
_NROWS = _B * _L

_NW = 32
_EPW = _NEDGE // _NW
_CHUNK = 256
_GPC = _CHUNK // 16
_NCHUNK = _EPW // _CHUNK

_LN10K = math.log(10000.0) / 64.0


def _prep_body(x_ref, w_ref, b_ref, ac_ref, we_ref, pew_ref):
    x = x_ref[...]

    def norm3(v):
        n = jnp.sqrt(jnp.sum(v * v, axis=0, keepdims=True))
        return v / jnp.maximum(n, 1e-12)

    n_at = x[0:3]
    ca = x[3:6]
    c_at = x[6:9]
    ca_n = norm3(n_at - ca)
    ca_c = norm3(c_at - ca)
    bis = norm3(ca_n + ca_c)
    perp = norm3(
        jnp.stack(
            [
                ca_n[1] * ca_c[2] - ca_n[2] * ca_c[1],
                ca_n[2] * ca_c[0] - ca_n[0] * ca_c[2],
                ca_n[0] * ca_c[1] - ca_n[1] * ca_c[0],
            ],
            axis=0,
        )
    )
    cb_dir = norm3(-bis + 0.5 * perp)
    vcb = ca + 1.54 * cb_dir
    ac_ref[...] = jnp.concatenate([x, vcb], axis=0)

    we_ref[...] = w_ref[0:_DWE, :]

    pos = lax.broadcasted_iota(jnp.int32, (65, 64), 0).astype(jnp.float32) - 32.0
    col = lax.broadcasted_iota(jnp.int32, (65, 64), 1)
    freq = ((col >> 1) * 2).astype(jnp.float32)
    ang = pos * jnp.exp(freq * (-_LN10K))
    pe = jnp.where((col % 2) == 0, jnp.sin(ang), jnp.cos(ang))
    pew_ref[...] = (
        jnp.dot(pe, w_ref[_DWE:, :], preferred_element_type=jnp.float32)
        + b_ref[...]
    )


def _sc_edges(coords_hbm, we_hbm, pew_hbm, nbr_hbm, out_hbm,
              coords_v, t_v, nbr_v, out_v, rb_v):
    wid = lax.axis_index("s") * 2 + lax.axis_index("c")
    ebase = wid * _EPW

    pltpu.sync_copy(coords_hbm, coords_v.at[pl.ds(0, 15 * _NROWS)])
    pltpu.sync_copy(we_hbm, t_v.at[pl.ds(0, _DWE * _DOUT)])
    pltpu.sync_copy(pew_hbm, t_v.at[pl.ds(_DWE * _DOUT, 65 * _DOUT)])
    pltpu.sync_copy(nbr_hbm.at[pl.ds(ebase, _EPW)], nbr_v)

    iota16 = lax.iota(jnp.int32, 16)
    b_id = ebase >> 14
    magic = jnp.int32(0x5F3759DF)

    def group_body(g, carry_g):
        e0 = ebase + g * 16
        l_id = (e0 >> 5) & (_L - 1)
        cb_row = (b_id << 9) + l_id
        nv = nbr_v[pl.ds(g * 16, 16)]
        n_row = nv + (b_id << 9)

        cc = [
            jnp.full((16,), coords_v[pl.ds(a * _NROWS + cb_row, 16)][0],
                     jnp.float32)
            for a in range(15)
        ]
        cn = [
            plsc.load_gather(coords_v, [n_row + a * _NROWS])
            for a in range(15)
        ]

        rbs = []
        for ai in range(_NATOMS):
            for aj in range(_NATOMS):
                p = ai * _NATOMS + aj
                dx = cc[3 * ai + 0] - cn[3 * aj + 0]
                dy = cc[3 * ai + 1] - cn[3 * aj + 1]
                dz = cc[3 * ai + 2] - cn[3 * aj + 2]
                s = dx * dx + dy * dy + dz * dz
                s = jnp.maximum(s, 1.0)
                yi = magic - (lax.bitcast_convert_type(s, jnp.int32) >> 1)
                y = lax.bitcast_convert_type(yi, jnp.float32)
                h = s * 0.5
                y = y * (1.5 - h * y * y)
                y = y * (1.5 - h * y * y)
                y = y * (1.5 - h * y * y)
                d = s * y
                xf = 2.0 * d - 4.0
                t = xf.astype(jnp.int32)
                ceil = jnp.where(xf > t.astype(jnp.float32), t + 1, t)
                binp = jnp.clip(ceil, 0, _NBINS - 1)
                rbs.append((binp + p * _NBINS) << 6)
        rel = jnp.clip(nv - l_id, -32, 32)
        rbs.append((rel + (_DWE + 32)) << 6)

        tr_idx = iota16 * 33
        for r in range(26):
            plsc.store_scatter(rb_v, [tr_idx + r], rbs[r])

        obase = (g % _GPC) * 16 * _DOUT

        def edge_body(le, carry):
            ob = obase + le * _DOUT
            v0 = rb_v[pl.ds(le * 33, 16)]
            v1 = rb_v[pl.ds(le * 33 + 16, 16)]
            rows = [v0[r] for r in range(16)] + [v1[r] for r in range(10)]
            accs = [t_v[pl.ds(rows[0] + 16 * u, 16)] for u in range(4)]
            for r in range(1, 26):
                for u in range(4):
                    accs[u] = accs[u] + t_v[pl.ds(rows[r] + 16 * u, 16)]
            for u in range(4):
                out_v[pl.ds(ob + 16 * u, 16)] = accs[u]
            return carry

        lax.fori_loop(0, 16, edge_body, 0)
        return carry_g

    def chunk_body(ci, carry_c):
        lax.fori_loop(ci * _GPC, (ci + 1) * _GPC, group_body, 0)
        off = (ebase + ci * _CHUNK) * _DOUT
        pltpu.sync_copy(out_v, out_hbm.at[pl.ds(off, _CHUNK * _DOUT)])
        return carry_c

    lax.fori_loop(0, _NCHUNK, chunk_body, 0)


_sc_call = functools.partial(
    pl.kernel,
    mesh=plsc.VectorSubcoreMesh(core_axis_name="c", subcore_axis_name="s"),
    out_type=jax.ShapeDtypeStruct((_NEDGE * _DOUT,), jnp.float32),
    compiler_params=pltpu.CompilerParams(needs_layout_passes=False),
    scratch_types=[
        pltpu.VMEM((15 * _NROWS + 16,), jnp.float32),
        pltpu.VMEM(((_DWE + 65) * _DOUT,), jnp.float32),
        pltpu.VMEM((_EPW,), jnp.int32),
        pltpu.VMEM((_CHUNK * _DOUT,), jnp.float32),
        pltpu.VMEM((16 * 33,), jnp.int32),
    ],
)(_sc_edges)


def kernel(coordinates, neighbor_indices, W_edge, b_edge):
    B, L, _, _ = coordinates.shape
    k = neighbor_indices.shape[-1]
    x_t = coordinates.transpose(2, 3, 0, 1).reshape(12, B * L)
    ac, we_t, pew_t = pl.pallas_call(
        _prep_body,
        out_shape=[
            jax.ShapeDtypeStruct((15, B * L), jnp.float32),
            jax.ShapeDtypeStruct((_DWE, _DOUT), jnp.float32),
            jax.ShapeDtypeStruct((65, _DOUT), jnp.float32),
        ],
    )(x_t, W_edge, b_edge.reshape(1, _DOUT))
    out_flat = _sc_call(
        ac.reshape(-1),
        we_t.reshape(-1),
        pew_t.reshape(-1),
        neighbor_indices.reshape(-1).astype(jnp.int32),
    )
    return out_flat.reshape(B, L, k, _DOUT)


# speedup vs baseline: 1.3785x; 1.1700x over previous
import functools
import math

import jax
import jax.numpy as jnp
from jax import lax
from jax.experimental import pallas as pl
from jax.experimental.pallas import tpu as pltpu
from jax.experimental.pallas import tpu_sc as plsc

_B = 2
_L = 512
_K = 32
_NATOMS = 5
_NPAIR = _NATOMS * _NATOMS
_NBINS = 37
_DWE = _NPAIR * _NBINS
_DOUT = 64
_NEDGE = _B * _L * _K
_NROWS = _B * _L

_NW = 32
_EPW = _NEDGE // _NW
_CHUNK = 256
_GPC = _CHUNK // 16
_NCHUNK = _EPW // _CHUNK

_LN10K = math.log(10000.0) / 64.0


def _prep_body(x_ref, w_ref, b_ref, ac_ref, we_ref, pew_ref):
    x = x_ref[...]

    def norm3(v):
        n = jnp.sqrt(jnp.sum(v * v, axis=0, keepdims=True))
        return v / jnp.maximum(n, 1e-12)

    n_at = x[0:3]
    ca = x[3:6]
    c_at = x[6:9]
    ca_n = norm3(n_at - ca)
    ca_c = norm3(c_at - ca)
    bis = norm3(ca_n + ca_c)
    perp = norm3(
        jnp.stack(
            [
                ca_n[1] * ca_c[2] - ca_n[2] * ca_c[1],
                ca_n[2] * ca_c[0] - ca_n[0] * ca_c[2],
                ca_n[0] * ca_c[1] - ca_n[1] * ca_c[0],
            ],
            axis=0,
        )
    )
    cb_dir = norm3(-bis + 0.5 * perp)
    vcb = ca + 1.54 * cb_dir
    ac_ref[...] = jnp.concatenate([x, vcb], axis=0)

    we_ref[...] = w_ref[0:_DWE, :]

    pos = lax.broadcasted_iota(jnp.int32, (65, 64), 0).astype(jnp.float32) - 32.0
    col = lax.broadcasted_iota(jnp.int32, (65, 64), 1)
    freq = ((col >> 1) * 2).astype(jnp.float32)
    ang = pos * jnp.exp(freq * (-_LN10K))
    pe = jnp.where((col % 2) == 0, jnp.sin(ang), jnp.cos(ang))
    pew_ref[...] = (
        jnp.dot(pe, w_ref[_DWE:, :], preferred_element_type=jnp.float32)
        + b_ref[...]
    )


def _sc_edges(coords_hbm, t_hbm, nbr_hbm, out_hbm,
              coords_v, t_v, nbr_v, out_v, rb_v):
    wid = lax.axis_index("s") * 2 + lax.axis_index("c")
    ebase = wid * _EPW

    pltpu.sync_copy(coords_hbm, coords_v.at[pl.ds(0, 15 * _NROWS)])
    pltpu.sync_copy(t_hbm, t_v)
    pltpu.sync_copy(nbr_hbm.at[pl.ds(ebase, _EPW)], nbr_v)

    iota16 = lax.iota(jnp.int32, 16)
    b_id = ebase >> 14
    magic = jnp.int32(0x5F3759DF)

    def group_body(g, carry_g):
        e0 = ebase + g * 16
        l_id = (e0 >> 5) & (_L - 1)
        cb_row = (b_id << 9) + l_id
        nv = nbr_v[pl.ds(g * 16, 16)]
        n_row = nv + (b_id << 9)

        cc = [
            jnp.full((16,), coords_v[pl.ds(a * _NROWS + cb_row, 16)][0],
                     jnp.float32)
            for a in range(15)
        ]
        cn = [
            plsc.load_gather(coords_v, [n_row + a * _NROWS])
            for a in range(15)
        ]

        rbs = []
        for ai in range(_NATOMS):
            for aj in range(_NATOMS):
                p = ai * _NATOMS + aj
                dx = cc[3 * ai + 0] - cn[3 * aj + 0]
                dy = cc[3 * ai + 1] - cn[3 * aj + 1]
                dz = cc[3 * ai + 2] - cn[3 * aj + 2]
                s = dx * dx + dy * dy + dz * dz
                s = jnp.maximum(s, 1.0)
                yi = magic - (lax.bitcast_convert_type(s, jnp.int32) >> 1)
                y = lax.bitcast_convert_type(yi, jnp.float32)
                h = s * 0.5
                y = y * (1.5 - h * y * y)
                y = y * (1.5 - h * y * y)
                y = y * (1.5 - h * y * y)
                d = s * y
                xf = 2.0 * d - 4.0
                t = xf.astype(jnp.int32)
                ceil = jnp.where(xf > t.astype(jnp.float32), t + 1, t)
                binp = jnp.clip(ceil, 0, _NBINS - 1)
                rbs.append((binp + p * _NBINS) << 5)
        rel = jnp.clip(nv - l_id, -32, 32)
        rbs.append((rel + (_DWE + 32)) << 5)

        tr_idx = iota16 * 33
        for r in range(26):
            plsc.store_scatter(rb_v, [tr_idx + r], rbs[r])

        obase = (g % _GPC) * 16 * _DOUT

        def edge_body(le, carry):
            ob = obase + le * _DOUT
            v0 = rb_v[pl.ds(le * 33, 16)]
            v1 = rb_v[pl.ds(le * 33 + 16, 16)]
            rows = [v0[r] for r in range(16)] + [v1[r] for r in range(10)]

            def row_halves(rw):
                out = []
                for h in range(2):
                    w = t_v[pl.ds(rw + 16 * h, 16)]
                    bf = plsc.bitcast(w, jnp.bfloat16)
                    a, b = plsc.unpack(bf, format=plsc.PackFormat.INTERLEAVED)
                    out += [a, b]
                return out

            accs = row_halves(rows[0])
            for r in range(1, 26):
                hv = row_halves(rows[r])
                for u in range(4):
                    accs[u] = accs[u] + hv[u]
            for u in range(4):
                out_v[pl.ds(ob + 16 * u, 16)] = accs[u]
            return carry

        lax.fori_loop(0, 16, edge_body, 0)
        return carry_g

    def chunk_body(ci, carry_c):
        lax.fori_loop(ci * _GPC, (ci + 1) * _GPC, group_body, 0)
        off = (ebase + ci * _CHUNK) * _DOUT
        pltpu.sync_copy(out_v, out_hbm.at[pl.ds(off, _CHUNK * _DOUT)])
        return carry_c

    lax.fori_loop(0, _NCHUNK, chunk_body, 0)


_sc_call = functools.partial(
    pl.kernel,
    mesh=plsc.VectorSubcoreMesh(core_axis_name="c", subcore_axis_name="s"),
    out_type=jax.ShapeDtypeStruct((_NEDGE * _DOUT,), jnp.float32),
    compiler_params=pltpu.CompilerParams(needs_layout_passes=False),
    scratch_types=[
        pltpu.VMEM((15 * _NROWS + 16,), jnp.float32),
        pltpu.VMEM(((_DWE + 65) * (_DOUT // 2),), jnp.int32),
        pltpu.VMEM((_EPW,), jnp.int32),
        pltpu.VMEM((_CHUNK * _DOUT,), jnp.float32),
        pltpu.VMEM((16 * 33,), jnp.int32),
    ],
)(_sc_edges)


def kernel(coordinates, neighbor_indices, W_edge, b_edge):
    B, L, _, _ = coordinates.shape
    k = neighbor_indices.shape[-1]
    x_t = coordinates.transpose(2, 3, 0, 1).reshape(12, B * L)
    ac, we_t, pew_t = pl.pallas_call(
        _prep_body,
        out_shape=[
            jax.ShapeDtypeStruct((15, B * L), jnp.float32),
            jax.ShapeDtypeStruct((_DWE, _DOUT), jnp.float32),
            jax.ShapeDtypeStruct((65, _DOUT), jnp.float32),
        ],
    )(x_t, W_edge, b_edge.reshape(1, _DOUT))
    t_full = jnp.concatenate([we_t, pew_t], axis=0)
    halves = [
        jnp.stack(
            [t_full[:, 32 * h : 32 * h + 16], t_full[:, 32 * h + 16 : 32 * h + 32]],
            axis=-1,
        ).reshape(-1, 32)
        for h in range(2)
    ]
    t_bf = jnp.concatenate(halves, axis=1).astype(jnp.bfloat16)
    t_i32 = lax.bitcast_convert_type(
        t_bf.reshape(_DWE + 65, _DOUT // 2, 2), jnp.int32
    ).reshape(-1)
    out_flat = _sc_call(
        ac.reshape(-1),
        t_i32,
        neighbor_indices.reshape(-1).astype(jnp.int32),
    )
    return out_flat.reshape(B, L, k, _DOUT)
